# baseline (speedup 1.0000x reference)
import jax
import jax.numpy as jnp
from jax import lax
from jax.experimental import pallas as pl
from jax.experimental.pallas import tpu as pltpu
from jax.experimental.pallas import tpu_sc as plsc

N_NODES = 10000
N_EDGES = 160000
CH = 256

NC = 2
NS = 16
LANES = 16
HCH = CH // NC
CHUNK = 50
NCHT = N_EDGES // CHUNK
DEG_CHUNK = 125
DEG_NCHT = N_EDGES // DEG_CHUNK
DEG_CPT = DEG_NCHT // (NC * NS)
SCAT_CPT = NCHT // NS
NBUF = 4
DEG_ROWS = 10240
ACC_ROWS = 10112
ROWS_PT = ACC_ROWS // NS
OUT_CHUNKS = (64,) * 9 + (56,)
IDX_REFILL = 40
N_REFILL = SCAT_CPT // IDX_REFILL


def _mesh():
    return plsc.VectorSubcoreMesh(core_axis_name="c", subcore_axis_name="s")


def _deg_body(col2d_hbm, deg_hbm, c2d, ones, zbuf, dacc):
    c = lax.axis_index("c")
    s = lax.axis_index("s")
    wid = c * NS + s

    def zb(i, carry):
        zbuf[pl.ds(i * LANES, LANES)] = jnp.zeros((LANES,), jnp.float32)
        return carry

    lax.fori_loop(0, 640 // LANES, zb, 0)
    for j in range(128 // LANES):
        ones[pl.ds(j * LANES, LANES)] = jnp.ones((LANES,), jnp.float32)

    pltpu.sync_copy(zbuf, dacc.at[pl.ds(s * 640, 640)])
    plsc.subcore_barrier()

    pltpu.sync_copy(col2d_hbm.at[pl.ds(wid * DEG_CPT, DEG_CPT)], c2d)

    def scat(q, carry):
        pltpu.sync_copy(ones.at[pl.ds(0, DEG_CHUNK)], dacc.at[c2d.at[q]], add=True)
        return carry

    lax.fori_loop(0, DEG_CPT, scat, 0)
    plsc.subcore_barrier()

    pltpu.sync_copy(dacc.at[pl.ds(s * 640, 640)], zbuf)
    pltpu.sync_copy(zbuf, deg_hbm.at[pl.ds(c * DEG_ROWS + s * 640, 640)])


_deg_call = pl.kernel(
    _deg_body,
    out_type=jax.ShapeDtypeStruct((NC * DEG_ROWS,), jnp.float32),
    mesh=_mesh(),
    scratch_types=[
        pltpu.VMEM((DEG_CPT, DEG_CHUNK), jnp.int32),
        pltpu.VMEM((128,), jnp.float32),
        pltpu.VMEM((640,), jnp.float32),
        pltpu.VMEM_SHARED((DEG_ROWS,), jnp.float32),
    ],
)


def _scat_body(g_hbm, row2d_hbm, col2d_hbm, acc_hbm,
               r2d, c2d, st0, gb0, gb1, gb2, gb3, sacc,
               sem0, sem1, sem2, sem3):
    c = lax.axis_index("c")
    s = lax.axis_index("s")

    def zrow(i, carry):
        for j in range(HCH // LANES):
            st0[i, pl.ds(j * LANES, LANES)] = jnp.zeros((LANES,), jnp.float32)
        return carry

    lax.fori_loop(0, 64, zrow, 0)
    off = 0
    for n in OUT_CHUNKS:
        pltpu.sync_copy(st0.at[pl.ds(0, n)],
                        sacc.at[pl.ds(s * ROWS_PT + off, n)])
        off += n

    plsc.subcore_barrier()

    gbufs = [gb.at[pl.ds(0, CHUNK)] for gb in (gb0, gb1, gb2, gb3)]
    gsems = (sem0, sem1, sem2, sem3)
    csl = pl.ds(c * HCH, HCH)

    for r in range(N_REFILL):
        hb = s * SCAT_CPT + r * IDX_REFILL
        pltpu.sync_copy(row2d_hbm.at[pl.ds(hb, IDX_REFILL)], r2d)
        pltpu.sync_copy(col2d_hbm.at[pl.ds(hb, IDX_REFILL)], c2d)

        for k in range(NBUF):
            pltpu.async_copy(g_hbm.at[r2d.at[k], csl], gbufs[k], gsems[k])

        def quad(h, carry):
            q = NBUF * h
            for k in range(NBUF):
                pltpu.make_async_copy(g_hbm.at[r2d.at[0], csl],
                                      gbufs[k], gsems[k]).wait()
                pltpu.sync_copy(gbufs[k], sacc.at[c2d.at[q + k]], add=True)

                @pl.when(q + k + NBUF < IDX_REFILL)
                def _():
                    pltpu.async_copy(g_hbm.at[r2d.at[q + k + NBUF], csl],
                                     gbufs[k], gsems[k])

            return carry

        lax.fori_loop(0, IDX_REFILL // NBUF, quad, 0)

    plsc.subcore_barrier()

    off = 0
    for n in OUT_CHUNKS:
        base = s * ROWS_PT + off
        pltpu.sync_copy(sacc.at[pl.ds(base, n)], st0.at[pl.ds(0, n)])
        pltpu.sync_copy(st0.at[pl.ds(0, n)], acc_hbm.at[c, pl.ds(base, n)])
        off += n


_scat_call = pl.kernel(
    _scat_body,
    out_type=jax.ShapeDtypeStruct((NC, ACC_ROWS, HCH), jnp.float32),
    mesh=_mesh(),
    scratch_types=[
        pltpu.VMEM((IDX_REFILL, CHUNK), jnp.int32),
        pltpu.VMEM((IDX_REFILL, CHUNK), jnp.int32),
        pltpu.VMEM((64, HCH), jnp.float32),
        pltpu.VMEM((CHUNK, HCH), jnp.float32),
        pltpu.VMEM((CHUNK, HCH), jnp.float32),
        pltpu.VMEM((CHUNK, HCH), jnp.float32),
        pltpu.VMEM((CHUNK, HCH), jnp.float32),
        pltpu.VMEM_SHARED((ACC_ROWS, HCH), jnp.float32),
        pltpu.SemaphoreType.DMA,
        pltpu.SemaphoreType.DMA,
        pltpu.SemaphoreType.DMA,
        pltpu.SemaphoreType.DMA,
    ],
)


_BM = 1000
_NB = N_NODES // _BM


def _mm_body(dinv_ref, x_ref, w_ref, o_ref):
    o_ref[...] = dinv_ref[pl.program_id(0)][:, None] * jnp.dot(
        x_ref[...], w_ref[...], preferred_element_type=jnp.float32)


def _mm(dinv2, x, w):
    return pl.pallas_call(
        _mm_body,
        grid=(_NB,),
        in_specs=[
            pl.BlockSpec((_NB, _BM), lambda i: (0, 0)),
            pl.BlockSpec((_BM, CH), lambda i: (i, 0)),
            pl.BlockSpec((CH, CH), lambda i: (0, 0)),
        ],
        out_specs=pl.BlockSpec((_BM, CH), lambda i: (i, 0)),
        out_shape=jax.ShapeDtypeStruct((N_NODES, CH), jnp.float32),
    )(dinv2, x, w)


def _stats_body(dinv_ref, a0_ref, a1_ref, g_ref, b_ref,
                y_ref, cs_ref, css_ref):
    t = jnp.concatenate([a0_ref[0], a1_ref[0]], axis=1) + g_ref[...]
    y = dinv_ref[pl.program_id(0)][:, None] * t + b_ref[...]
    y_ref[...] = y

    @pl.when(pl.program_id(0) == 0)
    def _():
        cs_ref[...] = jnp.zeros_like(cs_ref)
        css_ref[...] = jnp.zeros_like(css_ref)

    cs_ref[...] += jnp.sum(y, axis=0, keepdims=True)
    css_ref[...] += jnp.sum(y * y, axis=0, keepdims=True)


def _stats(dinv2, acc3, g, b2):
    return pl.pallas_call(
        _stats_body,
        grid=(_NB,),
        in_specs=[
            pl.BlockSpec((_NB, _BM), lambda i: (0, 0)),
            pl.BlockSpec((1, _BM, HCH), lambda i: (0, i, 0)),
            pl.BlockSpec((1, _BM, HCH), lambda i: (1, i, 0)),
            pl.BlockSpec((_BM, CH), lambda i: (i, 0)),
            pl.BlockSpec((1, CH), lambda i: (0, 0)),
        ],
        out_specs=[
            pl.BlockSpec((_BM, CH), lambda i: (i, 0)),
            pl.BlockSpec((1, CH), lambda i: (0, 0)),
            pl.BlockSpec((1, CH), lambda i: (0, 0)),
        ],
        out_shape=[
            jax.ShapeDtypeStruct((N_NODES, CH), jnp.float32),
            jax.ShapeDtypeStruct((1, CH), jnp.float32),
            jax.ShapeDtypeStruct((1, CH), jnp.float32),
        ],
    )(dinv2, acc3, acc3, g, b2)


def _nr_body(y_ref, mu_ref, sc_ref, o_ref):
    o_ref[...] = jnp.maximum((y_ref[...] - mu_ref[...]) * sc_ref[0, 0], 0.0)


def _nr(y, mu, scale):
    return pl.pallas_call(
        _nr_body,
        grid=(_NB,),
        in_specs=[
            pl.BlockSpec((_BM, CH), lambda i: (i, 0)),
            pl.BlockSpec((1, CH), lambda i: (0, 0)),
            pl.BlockSpec((1, 1), lambda i: (0, 0)),
        ],
        out_specs=pl.BlockSpec((_BM, CH), lambda i: (i, 0)),
        out_shape=jax.ShapeDtypeStruct((N_NODES, CH), jnp.float32),
    )(y, mu, scale)


def kernel(x, edge_index, W, b):
    row = edge_index[0]
    col = edge_index[1]
    row2d = row.reshape(NCHT, CHUNK)
    col2d = col.reshape(NCHT, CHUNK)
    col2d_deg = col.reshape(DEG_NCHT, DEG_CHUNK)

    deg1 = _deg_call(col2d_deg)
    deg = deg1[:N_NODES] + deg1[DEG_ROWS:DEG_ROWS + N_NODES]
    dinv = lax.rsqrt(deg + 1.0).reshape(_NB, _BM)

    g = _mm(dinv, x, W)
    acc3 = _scat_call(g, row2d, col2d)

    y, cs, css = _stats(dinv, acc3, g, b[None, :])
    mu = cs / N_NODES
    s = jnp.sum(css) / N_NODES - jnp.sum(mu * mu)
    scale = lax.rsqrt(1e-5 + s).reshape(1, 1)
    return _nr(y, mu, scale)


# speedup vs baseline: 1.0642x; 1.0642x over previous
import jax
import jax.numpy as jnp
from jax import lax
from jax.experimental import pallas as pl
from jax.experimental.pallas import tpu as pltpu
from jax.experimental.pallas import tpu_sc as plsc

N_NODES = 10000
N_EDGES = 160000
CH = 256

NC = 2
NS = 16
LANES = 16
HCH = CH // NC
CHUNK = 125
NCHT = N_EDGES // CHUNK
DEG_CPT = NCHT // (NC * NS)
SCAT_CPT = NCHT // NS
DEG_ROWS = 10240
ACC_ROWS = 10112
ROWS_PT = ACC_ROWS // NS
OUT_CHUNKS = (128, 128, 128, 128, 120)
IDX_HALF = SCAT_CPT // 2
HALF_PAIRS = IDX_HALF // 2


def _mesh():
    return plsc.VectorSubcoreMesh(core_axis_name="c", subcore_axis_name="s")


def _deg_body(col2d_hbm, deg_hbm, c2d, ones, zbuf, dacc):
    c = lax.axis_index("c")
    s = lax.axis_index("s")
    wid = c * NS + s

    def zb(i, carry):
        zbuf[pl.ds(i * LANES, LANES)] = jnp.zeros((LANES,), jnp.float32)
        return carry

    lax.fori_loop(0, 640 // LANES, zb, 0)
    for j in range(CHUNK // LANES + 1):
        ones[pl.ds(j * LANES, LANES)] = jnp.ones((LANES,), jnp.float32)

    pltpu.sync_copy(zbuf, dacc.at[pl.ds(s * 640, 640)])
    plsc.subcore_barrier()

    pltpu.sync_copy(col2d_hbm.at[pl.ds(wid * DEG_CPT, DEG_CPT)], c2d)

    def scat(q, carry):
        pltpu.sync_copy(ones.at[pl.ds(0, CHUNK)], dacc.at[c2d.at[q]], add=True)
        return carry

    lax.fori_loop(0, DEG_CPT, scat, 0)
    plsc.subcore_barrier()

    pltpu.sync_copy(dacc.at[pl.ds(s * 640, 640)], zbuf)
    pltpu.sync_copy(zbuf, deg_hbm.at[pl.ds(c * DEG_ROWS + s * 640, 640)])


_deg_call = pl.kernel(
    _deg_body,
    out_type=jax.ShapeDtypeStruct((NC * DEG_ROWS,), jnp.float32),
    mesh=_mesh(),
    scratch_types=[
        pltpu.VMEM((DEG_CPT, CHUNK), jnp.int32),
        pltpu.VMEM((CHUNK + LANES - CHUNK % LANES,), jnp.float32),
        pltpu.VMEM((640,), jnp.float32),
        pltpu.VMEM_SHARED((DEG_ROWS,), jnp.float32),
    ],
)


def _scat_body(g_hbm, row2d_hbm, col2d_hbm, acc_hbm,
               r2d, c2d, st0, st1, sacc, sem0, sem1):
    c = lax.axis_index("c")
    s = lax.axis_index("s")

    def zrow(i, carry):
        for j in range(HCH // LANES):
            st0[i, pl.ds(j * LANES, LANES)] = jnp.zeros((LANES,), jnp.float32)
        return carry

    lax.fori_loop(0, 128, zrow, 0)
    off = 0
    for n in OUT_CHUNKS:
        pltpu.sync_copy(st0.at[pl.ds(0, n)],
                        sacc.at[pl.ds(s * ROWS_PT + off, n)])
        off += n

    plsc.subcore_barrier()

    g0 = st0.at[pl.ds(0, CHUNK)]
    g1 = st1.at[pl.ds(0, CHUNK)]

    for half in range(2):
        hb = s * SCAT_CPT + half * IDX_HALF
        pltpu.sync_copy(row2d_hbm.at[pl.ds(hb, IDX_HALF)], r2d)
        pltpu.sync_copy(col2d_hbm.at[pl.ds(hb, IDX_HALF)], c2d)

        csl = pl.ds(c * HCH, HCH)
        pltpu.async_copy(g_hbm.at[r2d.at[0], csl], g0, sem0)
        pltpu.async_copy(g_hbm.at[r2d.at[1], csl], g1, sem1)

        def pair(h, carry):
            q0 = 2 * h
            pltpu.make_async_copy(g_hbm.at[r2d.at[0], csl], g0, sem0).wait()
            pltpu.sync_copy(g0, sacc.at[c2d.at[q0]], add=True)

            @pl.when(q0 + 2 < IDX_HALF)
            def _():
                pltpu.async_copy(g_hbm.at[r2d.at[q0 + 2], csl], g0, sem0)

            pltpu.make_async_copy(g_hbm.at[r2d.at[0], csl], g1, sem1).wait()
            pltpu.sync_copy(g1, sacc.at[c2d.at[q0 + 1]], add=True)

            @pl.when(q0 + 3 < IDX_HALF)
            def _():
                pltpu.async_copy(g_hbm.at[r2d.at[q0 + 3], csl], g1, sem1)

            return carry

        lax.fori_loop(0, HALF_PAIRS, pair, 0)

    plsc.subcore_barrier()

    off = 0
    for n in OUT_CHUNKS:
        base = s * ROWS_PT + off
        pltpu.sync_copy(sacc.at[pl.ds(base, n)], st0.at[pl.ds(0, n)])
        pltpu.sync_copy(st0.at[pl.ds(0, n)], acc_hbm.at[c, pl.ds(base, n)])
        off += n


_scat_call = pl.kernel(
    _scat_body,
    out_type=jax.ShapeDtypeStruct((NC, ACC_ROWS, HCH), jnp.float32),
    mesh=_mesh(),
    scratch_types=[
        pltpu.VMEM((IDX_HALF, CHUNK), jnp.int32),
        pltpu.VMEM((IDX_HALF, CHUNK), jnp.int32),
        pltpu.VMEM((128, HCH), jnp.float32),
        pltpu.VMEM((128, HCH), jnp.float32),
        pltpu.VMEM_SHARED((ACC_ROWS, HCH), jnp.float32),
        pltpu.SemaphoreType.DMA,
        pltpu.SemaphoreType.DMA,
    ],
)


_BM = 1000
_NB = N_NODES // _BM


def _mm_body(dinv_ref, x_ref, w_ref, o_ref):
    o_ref[...] = dinv_ref[pl.program_id(0)][:, None] * jnp.dot(
        x_ref[...], w_ref[...], preferred_element_type=jnp.float32)


def _mm(dinv2, x, w):
    return pl.pallas_call(
        _mm_body,
        grid=(_NB,),
        in_specs=[
            pl.BlockSpec((_NB, _BM), lambda i: (0, 0)),
            pl.BlockSpec((_BM, CH), lambda i: (i, 0)),
            pl.BlockSpec((CH, CH), lambda i: (0, 0)),
        ],
        out_specs=pl.BlockSpec((_BM, CH), lambda i: (i, 0)),
        out_shape=jax.ShapeDtypeStruct((N_NODES, CH), jnp.float32),
    )(dinv2, x, w)


def _stats_body(dinv_ref, a0_ref, a1_ref, g_ref, b_ref,
                y_ref, cs_ref, css_ref):
    t = jnp.concatenate([a0_ref[0], a1_ref[0]], axis=1) + g_ref[...]
    y = dinv_ref[pl.program_id(0)][:, None] * t + b_ref[...]
    y_ref[...] = y

    @pl.when(pl.program_id(0) == 0)
    def _():
        cs_ref[...] = jnp.zeros_like(cs_ref)
        css_ref[...] = jnp.zeros_like(css_ref)

    cs_ref[...] += jnp.sum(y, axis=0, keepdims=True)
    css_ref[...] += jnp.sum(y * y, axis=0, keepdims=True)


def _stats(dinv2, acc3, g, b2):
    return pl.pallas_call(
        _stats_body,
        grid=(_NB,),
        in_specs=[
            pl.BlockSpec((_NB, _BM), lambda i: (0, 0)),
            pl.BlockSpec((1, _BM, HCH), lambda i: (0, i, 0)),
            pl.BlockSpec((1, _BM, HCH), lambda i: (1, i, 0)),
            pl.BlockSpec((_BM, CH), lambda i: (i, 0)),
            pl.BlockSpec((1, CH), lambda i: (0, 0)),
        ],
        out_specs=[
            pl.BlockSpec((_BM, CH), lambda i: (i, 0)),
            pl.BlockSpec((1, CH), lambda i: (0, 0)),
            pl.BlockSpec((1, CH), lambda i: (0, 0)),
        ],
        out_shape=[
            jax.ShapeDtypeStruct((N_NODES, CH), jnp.float32),
            jax.ShapeDtypeStruct((1, CH), jnp.float32),
            jax.ShapeDtypeStruct((1, CH), jnp.float32),
        ],
    )(dinv2, acc3, acc3, g, b2)


def _nr_body(y_ref, cs_ref, css_ref, o_ref):
    mu = cs_ref[...] / N_NODES
    sval = jnp.sum(css_ref[...]) / N_NODES - jnp.sum(mu * mu)
    scale = lax.rsqrt(1e-5 + sval)
    o_ref[...] = jnp.maximum((y_ref[...] - mu) * scale, 0.0)


def _nr(y, cs, css):
    return pl.pallas_call(
        _nr_body,
        grid=(_NB,),
        in_specs=[
            pl.BlockSpec((_BM, CH), lambda i: (i, 0)),
            pl.BlockSpec((1, CH), lambda i: (0, 0)),
            pl.BlockSpec((1, CH), lambda i: (0, 0)),
        ],
        out_specs=pl.BlockSpec((_BM, CH), lambda i: (i, 0)),
        out_shape=jax.ShapeDtypeStruct((N_NODES, CH), jnp.float32),
    )(y, cs, css)


def kernel(x, edge_index, W, b):
    row = edge_index[0]
    col = edge_index[1]
    row2d = row.reshape(NCHT, CHUNK)
    col2d = col.reshape(NCHT, CHUNK)

    deg1 = _deg_call(col2d)
    deg = deg1[:N_NODES] + deg1[DEG_ROWS:DEG_ROWS + N_NODES]
    dinv = lax.rsqrt(deg + 1.0).reshape(_NB, _BM)

    g = _mm(dinv, x, W)
    acc3 = _scat_call(g, row2d, col2d)

    y, cs, css = _stats(dinv, acc3, g, b[None, :])
    return _nr(y, cs, css)


# speedup vs baseline: 1.1051x; 1.0384x over previous
import jax
import jax.numpy as jnp
from jax import lax
from jax.experimental import pallas as pl
from jax.experimental.pallas import tpu as pltpu
from jax.experimental.pallas import tpu_sc as plsc

N_NODES = 10000
N_EDGES = 160000
CH = 256

NC = 2
NS = 16
LANES = 16
HCH = CH // NC
CHUNK = 125
NCHT = N_EDGES // CHUNK
DEG_CPT = NCHT // (NC * NS)
SCAT_CPT = NCHT // NS
DEG_ROWS = 10240
ACC_ROWS = 10112
ROWS_PT = ACC_ROWS // NS
OUT_CHUNKS = (128, 128, 128, 128, 120)
IDX_HALF = SCAT_CPT // 2
HALF_PAIRS = IDX_HALF // 2


def _mesh():
    return plsc.VectorSubcoreMesh(core_axis_name="c", subcore_axis_name="s")


def _deg_body(col2d_hbm, deg_hbm, c2d, ones, zbuf, dacc):
    c = lax.axis_index("c")
    s = lax.axis_index("s")
    wid = c * NS + s

    def zb(i, carry):
        zbuf[pl.ds(i * LANES, LANES)] = jnp.zeros((LANES,), jnp.float32)
        return carry

    lax.fori_loop(0, 640 // LANES, zb, 0)
    for j in range(CHUNK // LANES + 1):
        ones[pl.ds(j * LANES, LANES)] = jnp.ones((LANES,), jnp.float32)

    pltpu.sync_copy(zbuf, dacc.at[pl.ds(s * 640, 640)])
    plsc.subcore_barrier()

    pltpu.sync_copy(col2d_hbm.at[pl.ds(wid * DEG_CPT, DEG_CPT)], c2d)

    def scat(q, carry):
        pltpu.sync_copy(ones.at[pl.ds(0, CHUNK)], dacc.at[c2d.at[q]], add=True)
        return carry

    lax.fori_loop(0, DEG_CPT, scat, 0)
    plsc.subcore_barrier()

    pltpu.sync_copy(dacc.at[pl.ds(s * 640, 640)], zbuf)
    pltpu.sync_copy(zbuf, deg_hbm.at[pl.ds(c * DEG_ROWS + s * 640, 640)])


_deg_call = pl.kernel(
    _deg_body,
    out_type=jax.ShapeDtypeStruct((NC * DEG_ROWS,), jnp.float32),
    mesh=_mesh(),
    scratch_types=[
        pltpu.VMEM((DEG_CPT, CHUNK), jnp.int32),
        pltpu.VMEM((CHUNK + LANES - CHUNK % LANES,), jnp.float32),
        pltpu.VMEM((640,), jnp.float32),
        pltpu.VMEM_SHARED((DEG_ROWS,), jnp.float32),
    ],
)


def _scat_body(g_hbm, row2d_hbm, col2d_hbm, acc_hbm,
               r2d, c2d, st0, st1, sacc, sem0, sem1):
    c = lax.axis_index("c")
    s = lax.axis_index("s")

    def zrow(i, carry):
        for j in range(HCH // LANES):
            st0[i, pl.ds(j * LANES, LANES)] = jnp.zeros((LANES,), jnp.float32)
        return carry

    lax.fori_loop(0, 128, zrow, 0)
    off = 0
    for n in OUT_CHUNKS:
        pltpu.sync_copy(st0.at[pl.ds(0, n)],
                        sacc.at[pl.ds(s * ROWS_PT + off, n)])
        off += n

    plsc.subcore_barrier()

    g0 = st0.at[pl.ds(0, CHUNK)]
    g1 = st1.at[pl.ds(0, CHUNK)]

    for half in range(2):
        hb = s * SCAT_CPT + half * IDX_HALF
        pltpu.sync_copy(row2d_hbm.at[pl.ds(hb, IDX_HALF)], r2d)
        pltpu.sync_copy(col2d_hbm.at[pl.ds(hb, IDX_HALF)], c2d)

        csl = pl.ds(c * HCH, HCH)
        pltpu.async_copy(g_hbm.at[r2d.at[0], csl], g0, sem0)
        pltpu.async_copy(g_hbm.at[r2d.at[1], csl], g1, sem1)

        def pair(h, carry):
            q0 = 2 * h
            pltpu.make_async_copy(g_hbm.at[r2d.at[0], csl], g0, sem0).wait()
            pltpu.sync_copy(g0, sacc.at[c2d.at[q0]], add=True)

            @pl.when(q0 + 2 < IDX_HALF)
            def _():
                pltpu.async_copy(g_hbm.at[r2d.at[q0 + 2], csl], g0, sem0)

            pltpu.make_async_copy(g_hbm.at[r2d.at[0], csl], g1, sem1).wait()
            pltpu.sync_copy(g1, sacc.at[c2d.at[q0 + 1]], add=True)

            @pl.when(q0 + 3 < IDX_HALF)
            def _():
                pltpu.async_copy(g_hbm.at[r2d.at[q0 + 3], csl], g1, sem1)

            return carry

        lax.fori_loop(0, HALF_PAIRS, pair, 0)

    plsc.subcore_barrier()

    off = 0
    for n in OUT_CHUNKS:
        base = s * ROWS_PT + off
        pltpu.sync_copy(sacc.at[pl.ds(base, n)], st0.at[pl.ds(0, n)])
        pltpu.sync_copy(st0.at[pl.ds(0, n)], acc_hbm.at[c, pl.ds(base, n)])
        off += n


_scat_call = pl.kernel(
    _scat_body,
    out_type=jax.ShapeDtypeStruct((NC, ACC_ROWS, HCH), jnp.float32),
    mesh=_mesh(),
    scratch_types=[
        pltpu.VMEM((IDX_HALF, CHUNK), jnp.int32),
        pltpu.VMEM((IDX_HALF, CHUNK), jnp.int32),
        pltpu.VMEM((128, HCH), jnp.float32),
        pltpu.VMEM((128, HCH), jnp.float32),
        pltpu.VMEM_SHARED((ACC_ROWS, HCH), jnp.float32),
        pltpu.SemaphoreType.DMA,
        pltpu.SemaphoreType.DMA,
    ],
)


_BM = 2000
_NB = N_NODES // _BM


def _mm_body(dinv_ref, x_ref, w_ref, o_ref):
    o_ref[...] = dinv_ref[pl.program_id(0)][:, None] * jnp.dot(
        x_ref[...], w_ref[...], preferred_element_type=jnp.float32)


def _mm(dinv2, x, w):
    return pl.pallas_call(
        _mm_body,
        grid=(_NB,),
        in_specs=[
            pl.BlockSpec((_NB, _BM), lambda i: (0, 0)),
            pl.BlockSpec((_BM, CH), lambda i: (i, 0)),
            pl.BlockSpec((CH, CH), lambda i: (0, 0)),
        ],
        out_specs=pl.BlockSpec((_BM, CH), lambda i: (i, 0)),
        out_shape=jax.ShapeDtypeStruct((N_NODES, CH), jnp.float32),
    )(dinv2, x, w)


def _stats_body(dinv_ref, a0_ref, a1_ref, g_ref, b_ref,
                y_ref, cs_ref, css_ref):
    t = jnp.concatenate([a0_ref[0], a1_ref[0]], axis=1) + g_ref[...]
    y = dinv_ref[pl.program_id(0)][:, None] * t + b_ref[...]
    y_ref[...] = y

    @pl.when(pl.program_id(0) == 0)
    def _():
        cs_ref[...] = jnp.zeros_like(cs_ref)
        css_ref[...] = jnp.zeros_like(css_ref)

    cs_ref[...] += jnp.sum(y, axis=0, keepdims=True)
    css_ref[...] += jnp.sum(y * y, axis=0, keepdims=True)


def _stats(dinv2, acc3, g, b2):
    return pl.pallas_call(
        _stats_body,
        grid=(_NB,),
        in_specs=[
            pl.BlockSpec((_NB, _BM), lambda i: (0, 0)),
            pl.BlockSpec((1, _BM, HCH), lambda i: (0, i, 0)),
            pl.BlockSpec((1, _BM, HCH), lambda i: (1, i, 0)),
            pl.BlockSpec((_BM, CH), lambda i: (i, 0)),
            pl.BlockSpec((1, CH), lambda i: (0, 0)),
        ],
        out_specs=[
            pl.BlockSpec((_BM, CH), lambda i: (i, 0)),
            pl.BlockSpec((1, CH), lambda i: (0, 0)),
            pl.BlockSpec((1, CH), lambda i: (0, 0)),
        ],
        out_shape=[
            jax.ShapeDtypeStruct((N_NODES, CH), jnp.float32),
            jax.ShapeDtypeStruct((1, CH), jnp.float32),
            jax.ShapeDtypeStruct((1, CH), jnp.float32),
        ],
    )(dinv2, acc3, acc3, g, b2)


def _nr_body(y_ref, cs_ref, css_ref, o_ref):
    mu = cs_ref[...] / N_NODES
    sval = jnp.sum(css_ref[...]) / N_NODES - jnp.sum(mu * mu)
    scale = lax.rsqrt(1e-5 + sval)
    o_ref[...] = jnp.maximum((y_ref[...] - mu) * scale, 0.0)


def _nr(y, cs, css):
    return pl.pallas_call(
        _nr_body,
        grid=(_NB,),
        in_specs=[
            pl.BlockSpec((_BM, CH), lambda i: (i, 0)),
            pl.BlockSpec((1, CH), lambda i: (0, 0)),
            pl.BlockSpec((1, CH), lambda i: (0, 0)),
        ],
        out_specs=pl.BlockSpec((_BM, CH), lambda i: (i, 0)),
        out_shape=jax.ShapeDtypeStruct((N_NODES, CH), jnp.float32),
    )(y, cs, css)


def kernel(x, edge_index, W, b):
    row = edge_index[0]
    col = edge_index[1]
    row2d = row.reshape(NCHT, CHUNK)
    col2d = col.reshape(NCHT, CHUNK)

    deg1 = _deg_call(col2d)
    deg = deg1[:N_NODES] + deg1[DEG_ROWS:DEG_ROWS + N_NODES]
    dinv = lax.rsqrt(deg + 1.0).reshape(_NB, _BM)

    g = _mm(dinv, x, W)
    acc3 = _scat_call(g, row2d, col2d)

    y, cs, css = _stats(dinv, acc3, g, b[None, :])
    return _nr(y, cs, css)
